# Initial kernel scaffold; baseline (speedup 1.0000x reference)
import jax
import jax.numpy as jnp
from jax.experimental import pallas as pl


def kernel(e, u, batch_non_omni, act_offsets, W1, b1, W2, b2, W3, b3):
    raise NotImplementedError("write your pallas kernel here")


# speedup vs baseline: 4.6867x; 4.6867x over previous
import jax
import jax.numpy as jnp
from jax.experimental import pallas as pl

_N_BLOCK = 1000
_CHUNK = 1000
_NEG = -1e30
_IMAX = 2147483647


def _mlp_body(e_ref, w1_ref, b1_ref, w2_ref, b2_ref, w3_ref, b3_ref, out_ref):
    h = jnp.dot(e_ref[...], w1_ref[...], preferred_element_type=jnp.float32)
    h = jnp.maximum(h + b1_ref[...], 0.0)
    h = jnp.dot(h, w2_ref[...], preferred_element_type=jnp.float32)
    h = jnp.maximum(h + b2_ref[...], 0.0)
    out_ref[...] = jnp.dot(h, w3_ref[...], preferred_element_type=jnp.float32) + b3_ref[...]


def _seg_body(logits_ref, seg_ref, u_ref, offs_ref, lp_ref, act_ref):
    n = logits_ref.shape[0]
    nb = n // _CHUNK
    b = offs_ref.shape[1]
    sid = jax.lax.broadcasted_iota(jnp.int32, (1, b), 1)
    neg = jnp.float32(_NEG)

    def _chunk(c):
        sl = pl.ds(c * _CHUNK, _CHUNK)
        return sl, logits_ref[sl, :], seg_ref[sl, :] == sid

    def _p1(c, mx):
        _, lg, m = _chunk(c)
        return jnp.maximum(mx, jnp.max(jnp.where(m, lg, neg), axis=0, keepdims=True))

    segmax = jax.lax.fori_loop(0, nb, _p1, jnp.full((1, b), neg, jnp.float32))

    def _p2(c, sm):
        _, lg, m = _chunk(c)
        gmx = jnp.max(jnp.where(m, segmax, neg), axis=1, keepdims=True)
        ex = jnp.exp(lg - gmx)
        return sm + jnp.sum(jnp.where(m, ex, 0.0), axis=0, keepdims=True)

    segsum = jax.lax.fori_loop(0, nb, _p2, jnp.zeros((1, b), jnp.float32))
    seglogsum = jnp.log(segsum)

    def _p3(c, gm):
        sl, lg, m = _chunk(c)
        gmax_g = jnp.max(jnp.where(m, segmax, neg), axis=1, keepdims=True)
        gls_g = jnp.max(jnp.where(m, seglogsum, neg), axis=1, keepdims=True)
        lp = lg - gmax_g - gls_g
        lp_ref[sl, :] = lp
        gl = lp + (-jnp.log(-jnp.log(u_ref[sl, :])))
        return jnp.maximum(gm, jnp.max(jnp.where(m, gl, neg), axis=0, keepdims=True))

    glmax = jax.lax.fori_loop(0, nb, _p3, jnp.full((1, b), neg, jnp.float32))

    def _p4(c, cm):
        sl, _, m = _chunk(c)
        gl = lp_ref[sl, :] + (-jnp.log(-jnp.log(u_ref[sl, :])))
        glmax_g = jnp.max(jnp.where(m, glmax, neg), axis=1, keepdims=True)
        idx = jax.lax.broadcasted_iota(jnp.int32, (_CHUNK, 1), 0) + c * _CHUNK
        cand = jnp.where(m & (gl == glmax_g), idx, jnp.int32(_IMAX))
        return jnp.minimum(cm, jnp.min(cand, axis=0, keepdims=True))

    candmin = jax.lax.fori_loop(0, nb, _p4, jnp.full((1, b), _IMAX, jnp.int32))
    act_ref[...] = candmin - offs_ref[...]


def kernel(e, u, batch_non_omni, act_offsets, W1, b1, W2, b2, W3, b3):
    n, esz = e.shape
    h = W1.shape[1]
    bsz = act_offsets.shape[0]

    logits2d = pl.pallas_call(
        _mlp_body,
        grid=(n // _N_BLOCK,),
        in_specs=[
            pl.BlockSpec((_N_BLOCK, esz), lambda i: (i, 0)),
            pl.BlockSpec((esz, h), lambda i: (0, 0)),
            pl.BlockSpec((1, h), lambda i: (0, 0)),
            pl.BlockSpec((h, h), lambda i: (0, 0)),
            pl.BlockSpec((1, h), lambda i: (0, 0)),
            pl.BlockSpec((h, 1), lambda i: (0, 0)),
            pl.BlockSpec((1, 1), lambda i: (0, 0)),
        ],
        out_specs=pl.BlockSpec((_N_BLOCK, 1), lambda i: (i, 0)),
        out_shape=jax.ShapeDtypeStruct((n, 1), jnp.float32),
    )(e, W1, b1.reshape(1, -1), W2, b2.reshape(1, -1), W3, b3.reshape(1, 1))

    lp2d, act2d = pl.pallas_call(
        _seg_body,
        out_shape=(
            jax.ShapeDtypeStruct((n, 1), jnp.float32),
            jax.ShapeDtypeStruct((1, bsz), jnp.int32),
        ),
    )(
        logits2d,
        batch_non_omni.reshape(-1, 1),
        u.reshape(-1, 1),
        act_offsets.reshape(1, -1),
    )

    return logits2d.reshape(-1), lp2d.reshape(-1), act2d.reshape(-1)
